# baseline (speedup 1.0000x reference)
import functools

import jax
import jax.numpy as jnp
from jax import lax
from jax.experimental import pallas as pl
from jax.experimental.pallas import tpu as pltpu
from jax.experimental.pallas import tpu_sc as plsc

_LANE = 128
_NCLASS = 64


def kernel(seq_len, relative_embeddings):
    del seq_len
    two_max_len, embed = relative_embeddings.shape
    s = two_max_len // 2
    row_blocks = s * embed // _LANE
    g_rows = two_max_len * embed // _LANE + 2

    info = plsc.get_sparse_core_info()
    num_workers = info.num_cores * info.num_subcores
    classes_per_w = _NCLASS // num_workers
    rows_per_class = s // _NCLASS

    flat = relative_embeddings.reshape(-1)
    padded = jnp.pad(flat, (0, g_rows * _LANE + 1008 - flat.shape[0]))
    offs = [0] + [1024 - 16 * c for c in range(1, _NCLASS)]
    shifted = jnp.stack(
        [lax.dynamic_slice(padded, (o,), (g_rows * _LANE,)) for o in offs]
    ).reshape(_NCLASS, g_rows, _LANE)

    mesh = plsc.VectorSubcoreMesh(core_axis_name="c", subcore_axis_name="s")

    @functools.partial(
        pl.kernel,
        mesh=mesh,
        out_type=jax.ShapeDtypeStruct((s, row_blocks, _LANE), jnp.float32),
        scratch_types=[
            pltpu.VMEM((1, g_rows, _LANE), jnp.float32),
            pltpu.SemaphoreType.DMA,
        ],
    )
    def toeplitz_rows(shifted_hbm, out_hbm, table_v, sem):
        wid = lax.axis_index("s") * info.num_cores + lax.axis_index("c")

        for j in range(classes_per_w):
            c = classes_per_w * wid + j
            base_m = 8 * jnp.where(c == 0, 32, 31)
            pltpu.sync_copy(shifted_hbm.at[pl.ds(c, 1)], table_v)

            def fire(t, carry):
                i = _NCLASS * t + c
                m = base_m - 8 * t
                pltpu.async_copy(
                    table_v.at[:, pl.ds(m, row_blocks), :],
                    out_hbm.at[pl.ds(i, 1)],
                    sem,
                )
                return carry

            lax.fori_loop(0, rows_per_class, fire, 0)

            def drain(t, carry):
                i = _NCLASS * t + c
                pltpu.make_async_copy(
                    table_v.at[:, pl.ds(0, row_blocks), :],
                    out_hbm.at[pl.ds(i, 1)],
                    sem,
                ).wait()
                return carry

            lax.fori_loop(0, rows_per_class, drain, 0)

    out = toeplitz_rows(shifted)
    return out.reshape(s, s, embed)


# speedup vs baseline: 2.0549x; 2.0549x over previous
import functools

import jax
import jax.numpy as jnp
from jax import lax
from jax.experimental import pallas as pl
from jax.experimental.pallas import tpu as pltpu
from jax.experimental.pallas import tpu_sc as plsc

_LANE = 128


def kernel(seq_len, relative_embeddings):
    del seq_len
    two_max_len, embed = relative_embeddings.shape
    s = two_max_len // 2
    g_cols = two_max_len + _LANE

    info = plsc.get_sparse_core_info()
    num_workers = info.num_cores * info.num_subcores
    classes_per_w = _LANE // num_workers
    rows_per_class = s // _LANE

    table_t = jnp.pad(relative_embeddings.T, ((0, 0), (0, 2 * _LANE)))
    shifts = [(_LANE - c) % _LANE for c in range(_LANE)]
    shifted = jnp.stack([table_t[:, r:r + g_cols] for r in shifts])

    mesh = plsc.VectorSubcoreMesh(core_axis_name="c", subcore_axis_name="s")

    @functools.partial(
        pl.kernel,
        mesh=mesh,
        out_type=jax.ShapeDtypeStruct((s, embed, s), jnp.float32),
        scratch_types=[
            pltpu.VMEM((1, embed, g_cols), jnp.float32),
            pltpu.SemaphoreType.DMA,
        ],
    )
    def toeplitz_planes(shifted_hbm, out_hbm, table_v, sem):
        wid = lax.axis_index("s") * info.num_cores + lax.axis_index("c")

        for j in range(classes_per_w):
            c = classes_per_w * wid + j
            r = (_LANE - c) % _LANE
            pltpu.sync_copy(shifted_hbm.at[pl.ds(c, 1)], table_v)

            def fire(k, carry):
                i = c + _LANE * k
                a = pl.multiple_of(s - i - r, _LANE)
                pltpu.async_copy(
                    table_v.at[:, :, pl.ds(a, s)],
                    out_hbm.at[pl.ds(i, 1)],
                    sem,
                )
                return carry

            lax.fori_loop(0, rows_per_class, fire, 0)

            def drain(k, carry):
                i = c + _LANE * k
                pltpu.make_async_copy(
                    table_v.at[:, :, pl.ds(0, s)],
                    out_hbm.at[pl.ds(i, 1)],
                    sem,
                ).wait()
                return carry

            lax.fori_loop(0, rows_per_class, drain, 0)

    return toeplitz_planes(shifted).transpose(0, 2, 1)


# speedup vs baseline: 3.3919x; 1.6506x over previous
import functools

import jax
import jax.numpy as jnp
from jax import lax
from jax.experimental import pallas as pl
from jax.experimental.pallas import tpu as pltpu
from jax.experimental.pallas import tpu_sc as plsc

_LANE = 128


def kernel(seq_len, relative_embeddings):
    del seq_len
    two_max_len, embed = relative_embeddings.shape
    s = two_max_len // 2
    g_cols = two_max_len + _LANE

    info = plsc.get_sparse_core_info()
    num_workers = info.num_cores * info.num_subcores
    classes_per_w = _LANE // num_workers
    rows_per_class = s // _LANE

    pad_cols = two_max_len + 2 * _LANE
    table_t = jnp.pad(relative_embeddings.T, ((0, 0), (0, 2 * _LANE)))

    def build_shifted(tab_ref, out_ref):
        c = pl.program_id(0)
        r = jnp.remainder(_LANE - c, _LANE)
        shift = jnp.remainder(pad_cols - r, pad_cols)
        rolled = pltpu.roll(tab_ref[...], shift, 1)
        out_ref[0] = rolled[:, :g_cols]

    shifted = pl.pallas_call(
        build_shifted,
        grid=(_LANE,),
        in_specs=[pl.BlockSpec((embed, pad_cols), lambda c: (0, 0))],
        out_specs=pl.BlockSpec((1, embed, g_cols), lambda c: (c, 0, 0)),
        out_shape=jax.ShapeDtypeStruct((_LANE, embed, g_cols), jnp.float32),
    )(table_t)

    mesh = plsc.VectorSubcoreMesh(core_axis_name="c", subcore_axis_name="s")

    @functools.partial(
        pl.kernel,
        mesh=mesh,
        out_type=jax.ShapeDtypeStruct((s, embed, s), jnp.float32),
        scratch_types=[
            pltpu.VMEM((1, embed, g_cols), jnp.float32),
            pltpu.SemaphoreType.DMA,
        ],
    )
    def toeplitz_planes(shifted_hbm, out_hbm, table_v, sem):
        wid = lax.axis_index("s") * info.num_cores + lax.axis_index("c")

        for j in range(classes_per_w):
            c = classes_per_w * wid + j
            r = (_LANE - c) % _LANE
            pltpu.sync_copy(shifted_hbm.at[pl.ds(c, 1)], table_v)

            def fire(k, carry):
                i = c + _LANE * k
                a = pl.multiple_of(s - i - r, _LANE)
                pltpu.async_copy(
                    table_v.at[:, :, pl.ds(a, s)],
                    out_hbm.at[pl.ds(i, 1)],
                    sem,
                )
                return carry

            lax.fori_loop(0, rows_per_class, fire, 0)

            def drain(k, carry):
                i = c + _LANE * k
                pltpu.make_async_copy(
                    table_v.at[:, :, pl.ds(0, s)],
                    out_hbm.at[pl.ds(i, 1)],
                    sem,
                ).wait()
                return carry

            lax.fori_loop(0, rows_per_class, drain, 0)

    return toeplitz_planes(shifted).transpose(0, 2, 1)


# speedup vs baseline: 4.4553x; 1.3135x over previous
import functools

import jax
import jax.numpy as jnp
from jax import lax
from jax.experimental import pallas as pl
from jax.experimental.pallas import tpu as pltpu
from jax.experimental.pallas import tpu_sc as plsc

_LANE = 128


def kernel(seq_len, relative_embeddings):
    del seq_len
    two_max_len, embed = relative_embeddings.shape
    s = two_max_len // 2
    g_cols = two_max_len - _LANE

    info = plsc.get_sparse_core_info()
    num_workers = info.num_cores * info.num_subcores
    classes_per_w = _LANE // num_workers
    rows_per_class = s // _LANE

    table_t = relative_embeddings.T

    def build_shifted(tab_ref, out_ref):
        for c in range(_LANE):
            out_ref[c] = tab_ref[:, _LANE - c:two_max_len - c]

    shifted = pl.pallas_call(
        build_shifted,
        out_shape=jax.ShapeDtypeStruct((_LANE, embed, g_cols), jnp.float32),
    )(table_t)

    mesh = plsc.VectorSubcoreMesh(core_axis_name="c", subcore_axis_name="s")

    @functools.partial(
        pl.kernel,
        mesh=mesh,
        out_type=jax.ShapeDtypeStruct((s, embed, s), jnp.float32),
        scratch_types=[
            pltpu.VMEM((1, embed, g_cols), jnp.float32),
            pltpu.VMEM((1, embed, g_cols), jnp.float32),
            pltpu.SemaphoreType.DMA,
            pltpu.SemaphoreType.DMA,
        ],
    )
    def toeplitz_planes(shifted_hbm, out_hbm, table_a, table_b, stage_sem, sem):
        wid = lax.axis_index("s") * info.num_cores + lax.axis_index("c")
        bufs = (table_a, table_b)

        def stage(j):
            pltpu.async_copy(
                shifted_hbm.at[pl.ds(classes_per_w * wid + j, 1)],
                bufs[j % 2],
                stage_sem,
            )

        def stage_wait(j):
            pltpu.make_async_copy(
                shifted_hbm.at[pl.ds(0, 1)], bufs[j % 2], stage_sem
            ).wait()

        def planes(j, fire):
            c = classes_per_w * wid + j
            buf = bufs[j % 2]

            def body(k, carry):
                i = c + _LANE * k
                a = pl.multiple_of(s - _LANE - _LANE * k, _LANE)
                copy = pltpu.make_async_copy(
                    buf.at[:, :, pl.ds(a, s)], out_hbm.at[pl.ds(i, 1)], sem
                )
                if fire:
                    copy.start()
                else:
                    copy.wait()
                return carry

            lax.fori_loop(0, rows_per_class, body, 0)

        stage(0)
        for j in range(classes_per_w):
            stage_wait(j)
            planes(j, fire=True)
            if j >= 1:
                planes(j - 1, fire=False)
            if j + 1 < classes_per_w:
                stage(j + 1)
        planes(classes_per_w - 1, fire=False)

    return toeplitz_planes(shifted).transpose(0, 2, 1)
